# Initial kernel scaffold; baseline (speedup 1.0000x reference)
import jax
import jax.numpy as jnp
from jax.experimental import pallas as pl


def kernel(h, edge_index, edge_features, n, W1, b1, W2, b2):
    raise NotImplementedError("write your pallas kernel here")


# speedup vs baseline: 2.4748x; 2.4748x over previous
import jax
import jax.numpy as jnp
import numpy as np
from jax import lax
from jax.experimental import pallas as pl
from jax.experimental.pallas import tpu as pltpu
from jax.experimental.pallas import tpu_sc as plsc

N_NODES = 10000
D = 128
EF_DIM = 16
N_EDGES = 320000
NC, NS, L = 2, 16, 16
NW = NC * NS
EPW = N_EDGES // NW
C = 80
NCH = EPW // C
GROUPS = D // L
STRIPE = 624
DROWS = 80

_F = np.float32


def _gelu16(x):
    u = x * _F(0.7071067811865476)
    a = jnp.abs(u)
    t = _F(1.0) / (_F(1.0) + _F(0.3275911) * a)
    poly = t * (_F(0.254829592) + t * (_F(-0.284496736) + t * (
        _F(1.421413741) + t * (_F(-1.453152027) + t * _F(1.061405429)))))
    erf_abs = _F(1.0) - poly * jnp.exp(-(u * u))
    erf = jnp.where(u >= _F(0.0), erf_abs, -erf_abs)
    return _F(0.5) * x * (_F(1.0) + erf)


def _pq_body(h_ref, wa_ref, wb_ref, p_ref, q_ref):
    h = h_ref[...]
    p_ref[...] = jnp.dot(h, wa_ref[...], preferred_element_type=jnp.float32)
    q_ref[...] = jnp.dot(h, wb_ref[...], preferred_element_type=jnp.float32)


_pq_call = pl.pallas_call(
    _pq_body,
    grid=(5,),
    in_specs=[
        pl.BlockSpec((2000, D), lambda i: (i, 0)),
        pl.BlockSpec((D, D), lambda i: (0, 0)),
        pl.BlockSpec((D, D), lambda i: (0, 0)),
    ],
    out_specs=[
        pl.BlockSpec((2000, D), lambda i: (i, 0)),
        pl.BlockSpec((2000, D), lambda i: (i, 0)),
    ],
    out_shape=[jax.ShapeDtypeStruct((N_NODES, D), jnp.float32)] * 2,
)


def _e_body(ef_ref, wc_ref, b1_ref, e_ref):
    e_ref[...] = jnp.dot(ef_ref[...], wc_ref[...],
                         preferred_element_type=jnp.float32) + b1_ref[...]


_e_call = pl.pallas_call(
    _e_body,
    grid=(50,),
    in_specs=[
        pl.BlockSpec((6400, EF_DIM), lambda i: (i, 0)),
        pl.BlockSpec((EF_DIM, D), lambda i: (0, 0)),
        pl.BlockSpec((1, D), lambda i: (0, 0)),
    ],
    out_specs=pl.BlockSpec((6400, D), lambda i: (i, 0)),
    out_shape=jax.ShapeDtypeStruct((N_EDGES, D), jnp.float32),
)


def _out_body(a0_ref, a1_ref, w2_ref, deg_ref, b2_ref, resid_ref, o_ref):
    acc = a0_ref[...] + a1_ref[...]
    o_ref[...] = (jnp.dot(acc, w2_ref[...], preferred_element_type=jnp.float32)
                  + deg_ref[...] * b2_ref[...] + resid_ref[0, 0])


_out_call = pl.pallas_call(
    _out_body,
    grid=(5,),
    in_specs=[
        pl.BlockSpec((2000, D), lambda i: (i, 0)),
        pl.BlockSpec((2000, D), lambda i: (i, 0)),
        pl.BlockSpec((D, D), lambda i: (0, 0)),
        pl.BlockSpec((2000, 1), lambda i: (i, 0)),
        pl.BlockSpec((1, D), lambda i: (0, 0)),
        pl.BlockSpec(memory_space=pltpu.SMEM),
    ],
    out_specs=pl.BlockSpec((2000, D), lambda i: (i, 0)),
    out_shape=jax.ShapeDtypeStruct((N_NODES, D), jnp.float32),
)


def _sc_body(p_hbm, q_hbm, e_hbm, rows_hbm, cols_hbm, accs_hbm, degs_hbm,
             acc_sh, deg_sh, rbuf, cbuf, dbuf, pbuf, qbuf, ebuf, obuf, sem):
    cid = lax.axis_index("c")
    sid = lax.axis_index("s")
    wid = cid * NS + sid

    zero16 = jnp.zeros((L,), jnp.float32)
    ones16 = jnp.ones((L,), jnp.float32)
    lane = lax.iota(jnp.int32, L)

    @pl.loop(0, C)
    def _zero_pbuf(e):
        for k in range(GROUPS):
            pbuf[e, pl.ds(k * L, L)] = zero16

    @pl.loop(0, L)
    def _zero_obuf(e):
        for k in range(GROUPS):
            obuf[e, pl.ds(k * L, L)] = zero16

    row0 = sid * STRIPE

    @pl.loop(0, 7)
    def _zero_acc(k):
        pltpu.sync_copy(pbuf, acc_sh.at[pl.ds(row0 + k * C, C)])

    pltpu.sync_copy(pbuf.at[pl.ds(0, STRIPE - 7 * C)],
                    acc_sh.at[pl.ds(row0 + 7 * C, STRIPE - 7 * C)])

    @pl.when(sid == NS - 1)
    def _zero_tail():
        pltpu.sync_copy(pbuf.at[pl.ds(0, N_NODES - NS * STRIPE)],
                        acc_sh.at[pl.ds(NS * STRIPE, N_NODES - NS * STRIPE)])

    @pl.when(sid == 0)
    def _zero_deg():
        pltpu.sync_copy(pbuf, deg_sh)

    plsc.subcore_barrier()

    ebase = wid * EPW

    @pl.loop(0, NCH)
    def _chunk(ch):
        base = ebase + ch * C
        pltpu.sync_copy(rows_hbm.at[pl.ds(base, C)], rbuf)
        pltpu.sync_copy(cols_hbm.at[pl.ds(base, C)], cbuf)
        cp_p = pltpu.async_copy(p_hbm.at[rbuf], pbuf, sem)
        cp_q = pltpu.async_copy(q_hbm.at[cbuf], qbuf, sem)
        cp_e = pltpu.async_copy(e_hbm.at[pl.ds(base, C)], ebuf, sem)
        cp_p.wait()
        cp_q.wait()
        cp_e.wait()

        @pl.loop(0, C)
        def _edge(e):
            for k in range(GROUPS):
                o = k * L
                x = pbuf[e, pl.ds(o, L)] + qbuf[e, pl.ds(o, L)] + ebuf[e, pl.ds(o, L)]
                pbuf[e, pl.ds(o, L)] = _gelu16(x)

        pltpu.sync_copy(pbuf, acc_sh.at[rbuf], add=True)

        for j in range(C // L):
            rv = rbuf[pl.ds(j * L, L)]
            dbuf[...] = lax.shift_right_logical(rv, 7)
            plsc.store_scatter(obuf, [lane, rv & (D - 1)], ones16)
            pltpu.sync_copy(obuf, deg_sh.at[dbuf], add=True)
            plsc.store_scatter(obuf, [lane, rv & (D - 1)], zero16)

    plsc.subcore_barrier()
    pltpu.sync_copy(acc_sh.at[pl.ds(row0, STRIPE)],
                    accs_hbm.at[cid, pl.ds(row0, STRIPE)])

    @pl.when(sid == NS - 1)
    def _copy_tail():
        pltpu.sync_copy(acc_sh.at[pl.ds(NS * STRIPE, N_NODES - NS * STRIPE)],
                        accs_hbm.at[cid, pl.ds(NS * STRIPE, N_NODES - NS * STRIPE)])

    @pl.when(sid == 0)
    def _copy_deg():
        pltpu.sync_copy(deg_sh, degs_hbm.at[cid])


_sc_call = pl.kernel(
    _sc_body,
    out_type=(
        jax.ShapeDtypeStruct((NC, N_NODES, D), jnp.float32),
        jax.ShapeDtypeStruct((NC, DROWS, D), jnp.float32),
    ),
    mesh=plsc.VectorSubcoreMesh(core_axis_name="c", subcore_axis_name="s"),
    compiler_params=pltpu.CompilerParams(needs_layout_passes=False),
    scratch_types=[
        pltpu.VMEM_SHARED((N_NODES, D), jnp.float32),
        pltpu.VMEM_SHARED((DROWS, D), jnp.float32),
        pltpu.VMEM((C,), jnp.int32),
        pltpu.VMEM((C,), jnp.int32),
        pltpu.VMEM((L,), jnp.int32),
        pltpu.VMEM((C, D), jnp.float32),
        pltpu.VMEM((C, D), jnp.float32),
        pltpu.VMEM((C, D), jnp.float32),
        pltpu.VMEM((L, D), jnp.float32),
        pltpu.SemaphoreType.DMA,
    ],
)


def kernel(h, edge_index, edge_features, n, W1, b1, W2, b2):
    rows = edge_index[0].astype(jnp.int32)
    cols = edge_index[1].astype(jnp.int32)
    P, Q = _pq_call(h, W1[:D], W1[D:2 * D])
    E = _e_call(edge_features, W1[2 * D:], b1.reshape(1, D))
    accs, degs = _sc_call(P, Q, E, rows, cols)
    deg = (degs[0] + degs[1]).reshape(-1)[:N_NODES].reshape(N_NODES, 1)
    resid = (jnp.asarray(n) - N_NODES).astype(jnp.float32).reshape(1, 1)
    return _out_call(accs[0], accs[1], W2, deg, b2.reshape(1, D), resid)


# speedup vs baseline: 2.9531x; 1.1933x over previous
import jax
import jax.numpy as jnp
import numpy as np
from jax import lax
from jax.experimental import pallas as pl
from jax.experimental.pallas import tpu as pltpu
from jax.experimental.pallas import tpu_sc as plsc

N_NODES = 10000
D = 128
EF_DIM = 16
N_EDGES = 320000
NC, NS, L = 2, 16, 16
NW = NC * NS
EPW = N_EDGES // NW
C = 80
NCH = EPW // C
GROUPS = D // L
STRIPE = 624
DROWS = 80

_F = np.float32


def _gelu16(x):
    w = _F(0.044715) * (x * x) + _F(1.0)
    e = jnp.exp(_F(-1.5957691216057308) * x * w)
    return x / (_F(1.0) + e)


def _pq_body(h_ref, wa_ref, wb_ref, p_ref, q_ref):
    h = h_ref[...]
    p_ref[...] = jnp.dot(h, wa_ref[...], preferred_element_type=jnp.float32)
    q_ref[...] = jnp.dot(h, wb_ref[...], preferred_element_type=jnp.float32)


_pq_call = pl.pallas_call(
    _pq_body,
    grid=(5,),
    in_specs=[
        pl.BlockSpec((2000, D), lambda i: (i, 0)),
        pl.BlockSpec((D, D), lambda i: (0, 0)),
        pl.BlockSpec((D, D), lambda i: (0, 0)),
    ],
    out_specs=[
        pl.BlockSpec((2000, D), lambda i: (i, 0)),
        pl.BlockSpec((2000, D), lambda i: (i, 0)),
    ],
    out_shape=[jax.ShapeDtypeStruct((N_NODES, D), jnp.float32)] * 2,
)


def _e_body(ef_ref, wc_ref, b1_ref, e_ref):
    e_ref[...] = jnp.dot(ef_ref[...], wc_ref[...],
                         preferred_element_type=jnp.float32) + b1_ref[...]


_e_call = pl.pallas_call(
    _e_body,
    grid=(50,),
    in_specs=[
        pl.BlockSpec((6400, EF_DIM), lambda i: (i, 0)),
        pl.BlockSpec((EF_DIM, D), lambda i: (0, 0)),
        pl.BlockSpec((1, D), lambda i: (0, 0)),
    ],
    out_specs=pl.BlockSpec((6400, D), lambda i: (i, 0)),
    out_shape=jax.ShapeDtypeStruct((N_EDGES, D), jnp.float32),
)


def _out_body(a0_ref, a1_ref, w2_ref, deg_ref, b2_ref, resid_ref, o_ref):
    acc = a0_ref[...] + a1_ref[...]
    o_ref[...] = (jnp.dot(acc, w2_ref[...], preferred_element_type=jnp.float32)
                  + deg_ref[...] * b2_ref[...] + resid_ref[0, 0])


_out_call = pl.pallas_call(
    _out_body,
    grid=(5,),
    in_specs=[
        pl.BlockSpec((2000, D), lambda i: (i, 0)),
        pl.BlockSpec((2000, D), lambda i: (i, 0)),
        pl.BlockSpec((D, D), lambda i: (0, 0)),
        pl.BlockSpec((2000, 1), lambda i: (i, 0)),
        pl.BlockSpec((1, D), lambda i: (0, 0)),
        pl.BlockSpec(memory_space=pltpu.SMEM),
    ],
    out_specs=pl.BlockSpec((2000, D), lambda i: (i, 0)),
    out_shape=jax.ShapeDtypeStruct((N_NODES, D), jnp.float32),
)


def _sc_body(p_hbm, q_hbm, e_hbm, rows_hbm, cols_hbm, accs_hbm, degs_hbm,
             acc_sh, deg_sh, rbuf, cbuf, dbuf, pbuf, qbuf, ebuf, obuf, sem):
    cid = lax.axis_index("c")
    sid = lax.axis_index("s")
    wid = cid * NS + sid

    zero16 = jnp.zeros((L,), jnp.float32)
    ones16 = jnp.ones((L,), jnp.float32)
    lane = lax.iota(jnp.int32, L)

    @pl.loop(0, C)
    def _zero_pbuf(e):
        for k in range(GROUPS):
            pbuf[e, pl.ds(k * L, L)] = zero16

    @pl.loop(0, C)
    def _zero_obuf(e):
        for k in range(GROUPS):
            obuf[e, pl.ds(k * L, L)] = zero16

    row0 = sid * STRIPE

    @pl.loop(0, 7)
    def _zero_acc(k):
        pltpu.sync_copy(pbuf, acc_sh.at[pl.ds(row0 + k * C, C)])

    pltpu.sync_copy(pbuf.at[pl.ds(0, STRIPE - 7 * C)],
                    acc_sh.at[pl.ds(row0 + 7 * C, STRIPE - 7 * C)])

    @pl.when(sid == NS - 1)
    def _zero_tail():
        pltpu.sync_copy(pbuf.at[pl.ds(0, N_NODES - NS * STRIPE)],
                        acc_sh.at[pl.ds(NS * STRIPE, N_NODES - NS * STRIPE)])

    @pl.when(sid == 0)
    def _zero_deg():
        pltpu.sync_copy(pbuf, deg_sh)

    plsc.subcore_barrier()

    ebase = wid * EPW

    @pl.loop(0, NCH)
    def _chunk(ch):
        base = ebase + ch * C
        pltpu.sync_copy(rows_hbm.at[pl.ds(base, C)], rbuf)
        pltpu.sync_copy(cols_hbm.at[pl.ds(base, C)], cbuf)
        cp_p = pltpu.async_copy(p_hbm.at[rbuf], pbuf, sem)
        cp_q = pltpu.async_copy(q_hbm.at[cbuf], qbuf, sem)
        cp_e = pltpu.async_copy(e_hbm.at[pl.ds(base, C)], ebuf, sem)
        cp_p.wait()
        cp_q.wait()
        cp_e.wait()

        @pl.loop(0, C)
        def _edge(e):
            for k in range(GROUPS):
                o = k * L
                x = pbuf[e, pl.ds(o, L)] + qbuf[e, pl.ds(o, L)] + ebuf[e, pl.ds(o, L)]
                pbuf[e, pl.ds(o, L)] = _gelu16(x)

        pltpu.sync_copy(pbuf, acc_sh.at[rbuf], add=True)

        for j in range(C // L):
            rv = rbuf[pl.ds(j * L, L)]
            dbuf[pl.ds(j * L, L)] = lax.shift_right_logical(rv, 7)
            plsc.store_scatter(obuf, [j * L + lane, rv & (D - 1)], ones16)
        pltpu.sync_copy(obuf, deg_sh.at[dbuf], add=True)
        for j in range(C // L):
            rv = rbuf[pl.ds(j * L, L)]
            plsc.store_scatter(obuf, [j * L + lane, rv & (D - 1)], zero16)

    plsc.subcore_barrier()
    pltpu.sync_copy(acc_sh.at[pl.ds(row0, STRIPE)],
                    accs_hbm.at[cid, pl.ds(row0, STRIPE)])

    @pl.when(sid == NS - 1)
    def _copy_tail():
        pltpu.sync_copy(acc_sh.at[pl.ds(NS * STRIPE, N_NODES - NS * STRIPE)],
                        accs_hbm.at[cid, pl.ds(NS * STRIPE, N_NODES - NS * STRIPE)])

    @pl.when(sid == 0)
    def _copy_deg():
        pltpu.sync_copy(deg_sh, degs_hbm.at[cid])


_sc_call = pl.kernel(
    _sc_body,
    out_type=(
        jax.ShapeDtypeStruct((NC, N_NODES, D), jnp.float32),
        jax.ShapeDtypeStruct((NC, DROWS, D), jnp.float32),
    ),
    mesh=plsc.VectorSubcoreMesh(core_axis_name="c", subcore_axis_name="s"),
    compiler_params=pltpu.CompilerParams(needs_layout_passes=False),
    scratch_types=[
        pltpu.VMEM_SHARED((N_NODES, D), jnp.float32),
        pltpu.VMEM_SHARED((DROWS, D), jnp.float32),
        pltpu.VMEM((C,), jnp.int32),
        pltpu.VMEM((C,), jnp.int32),
        pltpu.VMEM((C,), jnp.int32),
        pltpu.VMEM((C, D), jnp.float32),
        pltpu.VMEM((C, D), jnp.float32),
        pltpu.VMEM((C, D), jnp.float32),
        pltpu.VMEM((C, D), jnp.float32),
        pltpu.SemaphoreType.DMA,
    ],
)


def kernel(h, edge_index, edge_features, n, W1, b1, W2, b2):
    rows = edge_index[0].astype(jnp.int32)
    cols = edge_index[1].astype(jnp.int32)
    P, Q = _pq_call(h, W1[:D], W1[D:2 * D])
    E = _e_call(edge_features, W1[2 * D:], b1.reshape(1, D))
    accs, degs = _sc_call(P, Q, E, rows, cols)
    deg = (degs[0] + degs[1]).reshape(-1)[:N_NODES].reshape(N_NODES, 1)
    resid = (jnp.asarray(n) - N_NODES).astype(jnp.float32).reshape(1, 1)
    return _out_call(accs[0], accs[1], W2, deg, b2.reshape(1, D), resid)
